# baseline (speedup 1.0000x reference)
import jax
import jax.numpy as jnp
from jax.experimental import pallas as pl
from jax.experimental.pallas import tpu as pltpu
from jax.experimental.pallas import tpu_sc as plsc

B, L = 64, 4096
D = 64
K = 1024
T = 512
TT = 1024
BC = 8
NT = BC * T
N_TOK = B * T
GRID = B // BC
GW = 128


def _dot(a, b):
    return jax.lax.dot_general(a.astype(jnp.bfloat16), b.astype(jnp.bfloat16),
                               (((1,), (0,)), ((), ())),
                               preferred_element_type=jnp.float32)


def _enc_body(p2_ref, w1_ref, b1_ref, v012_ref, b2_ref, cbt_ref, cbtf_ref,
              qr_ref, idx_ref, stats_ref, counts_ref, sumd_ref, cn_ref, g_ref):
    i = pl.program_id(0)
    bf16 = jnp.bfloat16

    @pl.when(i == 0)
    def _():
        c = cbtf_ref[...]
        cn_ref[...] = jnp.sum(c * c, axis=0, keepdims=True)
        g_ref[:, 0:1, :] = jnp.zeros((BC, 1, 128), bf16)
        g_ref[:, T + 1:T + 2, :] = jnp.zeros((BC, 1, 128), bf16)

    g = jax.nn.relu(_dot(p2_ref[...], w1_ref[...]) + b1_ref[...])
    g_ref[:, 1:T + 1, :] = g.astype(bf16).reshape(BC, T, 128)

    gp = g_ref[:, 0:T, :].reshape(NT, 128)
    gc = g_ref[:, 1:T + 1, :].reshape(NT, 128)
    gn = g_ref[:, 2:T + 2, :].reshape(NT, 128)
    zf = (b2_ref[...] + _dot(gp, v012_ref[0]) + _dot(gc, v012_ref[1])
          + _dot(gn, v012_ref[2]))

    scores = _dot(-2.0 * zf, cbt_ref[...])
    dist = cn_ref[...] + scores
    minv = jnp.min(dist, axis=1, keepdims=True)
    oh = (dist == minv).astype(jnp.float32)
    qri = _dot(oh, qr_ref[...])
    idx = jnp.minimum(4.0 * qri[:, 0:1] + qri[:, 1:2], float(K - 1))
    idx_ref[...] = idx.astype(jnp.int32).reshape(BC, T)
    zn = jnp.sum(zf * zf, axis=1, keepdims=True)
    step_sum = jnp.sum(minv + zn)
    ones8 = jnp.ones((8, NT), dtype=jnp.bfloat16)
    cpart = _dot(ones8, oh)[0:1]

    @pl.when(i == 0)
    def _():
        counts_ref[...] = cpart
        sumd_ref[0, 0] = step_sum

    @pl.when(i > 0)
    def _():
        counts_ref[...] += cpart
        sumd_ref[0, 0] += step_sum

    @pl.when(i == GRID - 1)
    def _():
        p = counts_ref[...] / float(N_TOK)
        ent = -jnp.sum(p * jnp.log(p + 1e-10))
        perp = jnp.exp(ent)
        res = sumd_ref[0, 0] / float(N_TOK * D)
        lane = jax.lax.broadcasted_iota(jnp.int32, (1, 128), 1)
        stats_ref[...] = (jnp.where(lane == 0, res, 0.0)
                          + jnp.where(lane == 1, 0.25 * res, 0.0)
                          + jnp.where(lane == 2, perp, 0.0))


def _dec_body(zq_in_ref, d1_ref, d1b_ref, v36_ref, d2b_ref, out_ref,
              zq_ref, hd_ref):
    i = pl.program_id(0)
    bf16 = jnp.bfloat16

    @pl.when(i == 0)
    def _():
        zq_ref[:, 0:1, :] = jnp.zeros((BC, 1, D), bf16)
        zq_ref[:, T + 1:T + 2, :] = jnp.zeros((BC, 1, D), bf16)
        hd_ref[:, 0:1, :] = jnp.zeros((BC, 1, 128), bf16)
        hd_ref[:, T + 1:T + 2, :] = jnp.zeros((BC, 1, 128), bf16)

    zq_ref[:, 1:T + 1, :] = zq_in_ref[...].astype(bf16)
    zqp = zq_ref[:, 0:T, :].reshape(NT, D)
    zqc = zq_ref[:, 1:T + 1, :].reshape(NT, D)
    zqn = zq_ref[:, 2:T + 2, :].reshape(NT, D)
    ev = jax.nn.relu(d1b_ref[...] + _dot(zqc, d1_ref[1]) + _dot(zqp, d1_ref[3]))
    od = jax.nn.relu(d1b_ref[...] + _dot(zqc, d1_ref[2]) + _dot(zqn, d1_ref[0]))
    hd_ref[:, 1:T + 1, 0:D] = ev.astype(bf16).reshape(BC, T, D)
    hd_ref[:, 1:T + 1, D:128] = od.astype(bf16).reshape(BC, T, D)

    hp = hd_ref[:, 0:T, :].reshape(NT, 128)
    hc = hd_ref[:, 1:T + 1, :].reshape(NT, 128)
    hn = hd_ref[:, 2:T + 2, :].reshape(NT, 128)
    y_ev = _dot(hp, v36_ref[0]) + _dot(hc, v36_ref[1])
    y_od = _dot(hc, v36_ref[2]) + _dot(hn, v36_ref[3])
    y8 = jnp.concatenate([y_ev, y_od], axis=1) + d2b_ref[...]
    out_ref[...] = y8.reshape(BC, T, 8)


def _sc_gather(codebook, idx):
    cb_pad = jnp.pad(codebook, ((0, 0), (0, 128 - D)))
    vector_mesh = plsc.VectorSubcoreMesh(core_axis_name="c",
                                         subcore_axis_name="s")

    @pl.kernel(out_type=jax.ShapeDtypeStruct((N_TOK, 128), jnp.float32),
               mesh=vector_mesh)
    def gather_kernel(cb_hbm, i_hbm, o_hbm):
        def body(i_vmem, o_vmem):
            pltpu.sync_copy(cb_hbm.at[i_vmem.at[0]], o_vmem)

        pltpu.emit_pipeline(
            body,
            grid=(N_TOK // GW,),
            in_specs=[pl.BlockSpec((1, GW), index_map=lambda i: (0, i))],
            out_specs=[pl.BlockSpec((GW, 128), index_map=lambda i: (i, 0))],
            core_axis_name=("c", "s"),
            dimension_semantics=(pltpu.PARALLEL,),
        )(i_hbm, o_hbm)

    return gather_kernel(cb_pad, idx.reshape(1, N_TOK))[:, :D]


def kernel(x, W1, b1, W2, b2, codebook, D1w, D1b, D2w, D2b):
    f32, bf16 = jnp.float32, jnp.bfloat16
    x_pad = jnp.pad(x, ((0, 0), (2, 2)))
    xr = x_pad.reshape(B, L // 4 + 1, 4)
    patches = jnp.concatenate([xr[:, :TT, :], xr[:, 1:TT + 1, :]], axis=-1)
    p2 = patches.reshape(B * T, 16).astype(bf16)

    w1m = W1[:, 0, :].T
    zd = jnp.zeros((8, D), f32)
    w1blk = jnp.concatenate([
        jnp.concatenate([w1m, zd], axis=1),
        jnp.concatenate([zd, w1m], axis=1)], axis=0).astype(bf16)
    b1g = jnp.concatenate([b1, b1]).reshape(1, 128)

    w2m = jnp.transpose(W2, (2, 1, 0))
    zdd = jnp.zeros((D, D), f32)
    v0 = jnp.concatenate([zdd, w2m[0]], axis=0)
    v1 = jnp.concatenate([w2m[1], w2m[2]], axis=0)
    v2 = jnp.concatenate([w2m[3], zdd], axis=0)
    v012 = jnp.stack([v0, v1, v2]).astype(bf16)

    cbt = codebook.T
    ks = jnp.arange(K)
    qr = jnp.stack([ks // 4, ks % 4], axis=1).astype(bf16)

    d1m = jnp.transpose(D1w, (2, 1, 0)).astype(bf16)
    d2 = D2w[0]
    zc = jnp.zeros((D, 2), f32)
    zd4 = jnp.zeros((D, 4), f32)
    a_m = jnp.concatenate([d2[:, 6:8], zc], axis=1)
    b_m = d2[:, 2:6]
    c_m = jnp.concatenate([zc, d2[:, 0:2]], axis=1)
    v3 = jnp.concatenate([zd4, a_m], axis=0)
    v4 = jnp.concatenate([b_m, c_m], axis=0)
    v5 = jnp.concatenate([a_m, b_m], axis=0)
    v6 = jnp.concatenate([c_m, zd4], axis=0)
    v36 = jnp.stack([v3, v4, v5, v6]).astype(bf16)
    d2bv = jnp.broadcast_to(D2b, (8,)).reshape(1, 8)

    full = lambda *s: pl.BlockSpec(s, lambda i: (0,) * len(s))
    idx, stats = pl.pallas_call(
        _enc_body,
        grid=(GRID,),
        in_specs=[
            pl.BlockSpec((NT, 16), lambda i: (i, 0)),
            full(16, 128), full(1, 128), full(3, 128, D), full(1, D),
            full(D, K), full(D, K), full(K, 2),
        ],
        out_specs=[
            pl.BlockSpec((BC, T), lambda i: (i, 0)),
            pl.BlockSpec((1, 128), lambda i: (0, 0)),
        ],
        out_shape=[
            jax.ShapeDtypeStruct((B, T), jnp.int32),
            jax.ShapeDtypeStruct((1, 128), f32),
        ],
        scratch_shapes=[
            pltpu.VMEM((1, K), f32),
            pltpu.SMEM((1, 1), f32),
            pltpu.VMEM((1, K), f32),
            pltpu.VMEM((BC, T + 2, 128), bf16),
        ],
    )(p2, w1blk, b1g, v012, b2.reshape(1, D), cbt.astype(bf16), cbt, qr)

    zq = _sc_gather(codebook, idx).reshape(B, T, D)

    out = pl.pallas_call(
        _dec_body,
        grid=(GRID,),
        in_specs=[
            pl.BlockSpec((BC, T, D), lambda i: (i, 0, 0)),
            full(4, D, D), full(1, D), full(4, 128, 4), full(1, 8),
        ],
        out_specs=pl.BlockSpec((BC, T, 8), lambda i: (i, 0, 0)),
        out_shape=jax.ShapeDtypeStruct((B, T, 8), f32),
        scratch_shapes=[
            pltpu.VMEM((BC, T + 2, D), bf16),
            pltpu.VMEM((BC, T + 2, 128), bf16),
        ],
    )(zq, d1m, D1b.reshape(1, D), v36, d2bv)

    x_recon = out.reshape(B, L)
    return (x_recon, stats[0, 1], stats[0, 2], stats[0, 0])


# speedup vs baseline: 2.2118x; 2.2118x over previous
import jax
import jax.numpy as jnp
from jax.experimental import pallas as pl
from jax.experimental.pallas import tpu as pltpu

B, L = 64, 4096
D = 64
K = 1024
T = 512
TT = 1024
BC = 8
NT = BC * T
N_TOK = B * T
GRID = B // BC


def _dot(a, b):
    return jax.lax.dot_general(a.astype(jnp.bfloat16), b.astype(jnp.bfloat16),
                               (((1,), (0,)), ((), ())),
                               preferred_element_type=jnp.float32)


def _vq_body(p2_ref, w1_ref, b1_ref, v012_ref, b2_ref, cbtm_ref,
             cb_ref, d1_ref, d1b_ref, v36_ref, d2b_ref,
             out_ref, stats_ref,
             counts_ref, sumd_ref, g_ref, zq_ref, hd_ref):
    i = pl.program_id(0)
    bf16 = jnp.bfloat16

    @pl.when(i == 0)
    def _():
        g_ref[:, 0:1, :] = jnp.zeros((BC, 1, 128), bf16)
        g_ref[:, T + 1:T + 2, :] = jnp.zeros((BC, 1, 128), bf16)
        zq_ref[:, 0:1, :] = jnp.zeros((BC, 1, D), bf16)
        zq_ref[:, T + 1:T + 2, :] = jnp.zeros((BC, 1, D), bf16)
        hd_ref[:, 0:1, :] = jnp.zeros((BC, 1, 128), bf16)
        hd_ref[:, T + 1:T + 2, :] = jnp.zeros((BC, 1, 128), bf16)

    g = jax.nn.relu(_dot(p2_ref[...], w1_ref[...]) + b1_ref[...])
    g_ref[:, 1:T + 1, :] = g.astype(bf16).reshape(BC, T, 128)

    gp = g_ref[:, 0:T, :].reshape(NT, 128)
    gc = g_ref[:, 1:T + 1, :].reshape(NT, 128)
    gn = g_ref[:, 2:T + 2, :].reshape(NT, 128)
    zf = (b2_ref[...] + _dot(gp, v012_ref[0]) + _dot(gc, v012_ref[1])
          + _dot(gn, v012_ref[2]))

    dist = _dot(zf, cbtm_ref[...])
    minv = jnp.min(dist, axis=1, keepdims=True)
    oh = (dist == minv).astype(jnp.float32)
    zq = _dot(oh, cb_ref[...])
    step_sum = (jnp.sum(minv) + jnp.sum(zf * zf)
                - 2.0 * NT)
    ones8 = jnp.ones((8, NT), dtype=jnp.bfloat16)
    cpart = _dot(ones8, oh)[0:1]

    @pl.when(i == 0)
    def _():
        counts_ref[...] = cpart
        sumd_ref[0, 0] = step_sum

    @pl.when(i > 0)
    def _():
        counts_ref[...] += cpart
        sumd_ref[0, 0] += step_sum

    zq_ref[:, 1:T + 1, :] = zq.astype(bf16).reshape(BC, T, D)
    zqp = zq_ref[:, 0:T, :].reshape(NT, D)
    zqc = zq_ref[:, 1:T + 1, :].reshape(NT, D)
    zqn = zq_ref[:, 2:T + 2, :].reshape(NT, D)
    ev = jax.nn.relu(d1b_ref[...] + _dot(zqc, d1_ref[1]) + _dot(zqp, d1_ref[3]))
    od = jax.nn.relu(d1b_ref[...] + _dot(zqc, d1_ref[2]) + _dot(zqn, d1_ref[0]))
    hd_ref[:, 1:T + 1, 0:D] = ev.astype(bf16).reshape(BC, T, D)
    hd_ref[:, 1:T + 1, D:128] = od.astype(bf16).reshape(BC, T, D)

    hp = hd_ref[:, 0:T, :].reshape(NT, 128)
    hc = hd_ref[:, 1:T + 1, :].reshape(NT, 128)
    hn = hd_ref[:, 2:T + 2, :].reshape(NT, 128)
    y_ev = _dot(hp, v36_ref[0]) + _dot(hc, v36_ref[1])
    y_od = _dot(hc, v36_ref[2]) + _dot(hn, v36_ref[3])
    y8 = jnp.concatenate([y_ev, y_od], axis=1) + d2b_ref[...]
    out_ref[...] = y8.reshape(BC, T, 8)

    @pl.when(i == GRID - 1)
    def _():
        p = counts_ref[...] / float(N_TOK)
        ent = -jnp.sum(p * jnp.log(p + 1e-10))
        perp = jnp.exp(ent)
        res = sumd_ref[0, 0] / float(N_TOK * D)
        lane = jax.lax.broadcasted_iota(jnp.int32, (1, 128), 1)
        stats_ref[...] = (jnp.where(lane == 0, res, 0.0)
                          + jnp.where(lane == 1, 0.25 * res, 0.0)
                          + jnp.where(lane == 2, perp, 0.0))


def kernel(x, W1, b1, W2, b2, codebook, D1w, D1b, D2w, D2b):
    f32, bf16 = jnp.float32, jnp.bfloat16
    x_pad = jnp.pad(x, ((0, 0), (2, 2)))
    xr = x_pad.reshape(B, L // 4 + 1, 4)
    patches = jnp.concatenate([xr[:, :TT, :], xr[:, 1:TT + 1, :]], axis=-1)
    p2 = patches.reshape(B * T, 16).astype(bf16)

    w1m = W1[:, 0, :].T
    zd = jnp.zeros((8, D), f32)
    w1blk = jnp.concatenate([
        jnp.concatenate([w1m, zd], axis=1),
        jnp.concatenate([zd, w1m], axis=1)], axis=0).astype(bf16)
    b1g = jnp.concatenate([b1, b1]).reshape(1, 128)

    w2m = jnp.transpose(W2, (2, 1, 0))
    zdd = jnp.zeros((D, D), f32)
    v0 = jnp.concatenate([zdd, w2m[0]], axis=0)
    v1 = jnp.concatenate([w2m[1], w2m[2]], axis=0)
    v2 = jnp.concatenate([w2m[3], zdd], axis=0)
    v012 = jnp.pad(jnp.stack([v0, v1, v2]),
                   ((0, 0), (0, 0), (0, 2))).astype(bf16)
    b2aug = jnp.concatenate([b2, jnp.ones((2,), f32)]).reshape(1, D + 2)

    cbt = codebook.T
    cn = jnp.sum(codebook * codebook, axis=1)
    cn_hi = cn.astype(bf16).astype(f32)
    cn_lo = cn - cn_hi
    cbtm = jnp.concatenate([-2.0 * cbt, cn_hi[None, :], cn_lo[None, :]],
                           axis=0).astype(bf16)
    d1m = jnp.transpose(D1w, (2, 1, 0)).astype(bf16)
    d2 = D2w[0]
    zc = jnp.zeros((D, 2), f32)
    zd4 = jnp.zeros((D, 4), f32)
    a_m = jnp.concatenate([d2[:, 6:8], zc], axis=1)
    b_m = d2[:, 2:6]
    c_m = jnp.concatenate([zc, d2[:, 0:2]], axis=1)
    v3 = jnp.concatenate([zd4, a_m], axis=0)
    v4 = jnp.concatenate([b_m, c_m], axis=0)
    v5 = jnp.concatenate([a_m, b_m], axis=0)
    v6 = jnp.concatenate([c_m, zd4], axis=0)
    v36 = jnp.stack([v3, v4, v5, v6]).astype(bf16)
    d2bv = jnp.broadcast_to(D2b, (8,)).reshape(1, 8)

    full = lambda *s: pl.BlockSpec(s, lambda i: (0,) * len(s))
    out, stats = pl.pallas_call(
        _vq_body,
        grid=(GRID,),
        in_specs=[
            pl.BlockSpec((NT, 16), lambda i: (i, 0)),
            full(16, 128), full(1, 128), full(3, 128, D + 2), full(1, D + 2),
            full(D + 2, K), full(K, D),
            full(4, D, D), full(1, D),
            full(4, 128, 4), full(1, 8),
        ],
        out_specs=[
            pl.BlockSpec((BC, T, 8), lambda i: (i, 0, 0)),
            pl.BlockSpec((1, 128), lambda i: (0, 0)),
        ],
        out_shape=[
            jax.ShapeDtypeStruct((B, T, 8), f32),
            jax.ShapeDtypeStruct((1, 128), f32),
        ],
        scratch_shapes=[
            pltpu.VMEM((1, K), f32),
            pltpu.SMEM((1, 1), f32),
            pltpu.VMEM((BC, T + 2, 128), bf16),
            pltpu.VMEM((BC, T + 2, D), bf16),
            pltpu.VMEM((BC, T + 2, 128), bf16),
        ],
    )(p2, w1blk, b1g, v012, b2aug,
      cbtm, codebook.astype(bf16),
      d1m, D1b.reshape(1, D), v36, d2bv)

    x_recon = out.reshape(B, L)
    return (x_recon, stats[0, 1], stats[0, 2], stats[0, 0])
